# baseline (speedup 1.0000x reference)
import jax
import jax.numpy as jnp
from jax.experimental import pallas as pl

_BETA = 0.5
_ALPHA = 0.2


def _gat_block_kernel(inp_ref, w_ref, a_ref, adj_ref, m_ref, out_ref,
                      h_ref, fg_ref, *, block_rows):
    i = pl.program_id(0)

    @pl.when(i == 0)
    def _prologue():
        h = jnp.dot(inp_ref[:], w_ref[:], preferred_element_type=jnp.float32)
        h_ref[:] = h
        dout = h.shape[1]
        a1 = a_ref[0, :dout]
        a2 = a_ref[0, dout:]
        fg_ref[0, :] = jnp.dot(h, a1, preferred_element_type=jnp.float32)
        fg_ref[1, :] = jnp.dot(h, a2, preferred_element_type=jnp.float32)

    f_blk = fg_ref[0, pl.ds(i * block_rows, block_rows)]
    g = fg_ref[1, :]
    adj_blk = adj_ref[:]
    mb = _BETA * m_ref[:] + (1.0 - _BETA) * adj_blk
    logit = f_blk[:, None] + mb * g[None, :]
    e = adj_blk * jnp.exp(jnp.where(logit >= 0, logit, _ALPHA * logit))
    rowsum = jnp.sum(e, axis=1, keepdims=True)
    hp = jnp.dot(e, h_ref[:], preferred_element_type=jnp.float32) / rowsum
    out_ref[:] = jnp.where(hp > 0, hp, jnp.exp(jnp.minimum(hp, 0.0)) - 1.0)


def kernel(input, adj, M, W, a):
    N, din = input.shape
    dout = W.shape[1]
    block_rows = 256
    grid = (N // block_rows,)
    from functools import partial
    from jax.experimental.pallas import tpu as pltpu
    return pl.pallas_call(
        partial(_gat_block_kernel, block_rows=block_rows),
        grid=grid,
        in_specs=[
            pl.BlockSpec((N, din), lambda i: (0, 0)),
            pl.BlockSpec((din, dout), lambda i: (0, 0)),
            pl.BlockSpec((1, 2 * dout), lambda i: (0, 0)),
            pl.BlockSpec((block_rows, N), lambda i: (i, 0)),
            pl.BlockSpec((block_rows, N), lambda i: (i, 0)),
        ],
        out_specs=pl.BlockSpec((block_rows, dout), lambda i: (i, 0)),
        out_shape=jax.ShapeDtypeStruct((N, dout), jnp.float32),
        scratch_shapes=[
            pltpu.VMEM((N, dout), jnp.float32),
            pltpu.VMEM((2, N), jnp.float32),
        ],
    )(input, W, a, adj, M)


# speedup vs baseline: 1.0797x; 1.0797x over previous
from functools import partial

import jax
import jax.numpy as jnp
from jax.experimental import pallas as pl
from jax.experimental.pallas import tpu as pltpu

_BETA = 0.5
_ALPHA = 0.2


def _gat_block_kernel(inp_ref, inp_blk_ref, w_ref, a_ref, adj_ref, m_ref, out_ref):
    h = jnp.dot(inp_ref[:], w_ref[:], preferred_element_type=jnp.float32)
    dout = h.shape[1]
    a1 = a_ref[0, :dout]
    a2 = a_ref[0, dout:]
    g = jnp.dot(h, a2, preferred_element_type=jnp.float32)
    h_blk = jnp.dot(inp_blk_ref[:], w_ref[:], preferred_element_type=jnp.float32)
    f_blk = jnp.dot(h_blk, a1, preferred_element_type=jnp.float32)

    adj_blk = adj_ref[:]
    mb = _BETA * m_ref[:] + (1.0 - _BETA) * adj_blk
    logit = f_blk[:, None] + mb * g[None, :]
    e = adj_blk * jnp.exp(jnp.where(logit >= 0, logit, _ALPHA * logit))
    rowsum = jnp.sum(e, axis=1, keepdims=True)
    hp = jnp.dot(e, h, preferred_element_type=jnp.float32) / rowsum
    out_ref[:] = jnp.where(hp > 0, hp, jnp.exp(jnp.minimum(hp, 0.0)) - 1.0)


def kernel(input, adj, M, W, a):
    N, din = input.shape
    dout = W.shape[1]
    block_rows = 1024
    grid = (N // block_rows,)
    return pl.pallas_call(
        _gat_block_kernel,
        grid=grid,
        in_specs=[
            pl.BlockSpec((N, din), lambda i: (0, 0)),
            pl.BlockSpec((block_rows, din), lambda i: (i, 0)),
            pl.BlockSpec((din, dout), lambda i: (0, 0)),
            pl.BlockSpec((1, 2 * dout), lambda i: (0, 0)),
            pl.BlockSpec((block_rows, N), lambda i: (i, 0)),
            pl.BlockSpec((block_rows, N), lambda i: (i, 0)),
        ],
        out_specs=pl.BlockSpec((block_rows, dout), lambda i: (i, 0)),
        out_shape=jax.ShapeDtypeStruct((N, dout), jnp.float32),
    )(input, input, W, a, adj, M)


# speedup vs baseline: 1.1670x; 1.0809x over previous
from functools import partial

import jax
import jax.numpy as jnp
from jax.experimental import pallas as pl
from jax.experimental.pallas import tpu as pltpu

_BETA = 0.5
_ALPHA = 0.2


def _gat_block_kernel(inp_ref, inp_blk_ref, w_ref, a_ref, adj_ref, m_ref, out_ref):
    h = jnp.dot(inp_ref[:], w_ref[:], preferred_element_type=jnp.float32)
    dout = h.shape[1]
    a1 = a_ref[0, :dout]
    a2 = a_ref[0, dout:]
    g = jnp.dot(h, a2, preferred_element_type=jnp.float32)
    h_blk = jnp.dot(inp_blk_ref[:], w_ref[:], preferred_element_type=jnp.float32)
    f_blk = jnp.dot(h_blk, a1, preferred_element_type=jnp.float32)

    adj_blk = adj_ref[:]
    mb = _BETA * m_ref[:] + (1.0 - _BETA) * adj_blk
    logit = f_blk[:, None] + mb * g[None, :]
    e = adj_blk * jnp.exp(jnp.where(logit >= 0, logit, _ALPHA * logit))
    rowsum = jnp.sum(e, axis=1, keepdims=True)
    hp = jnp.dot(e, h, preferred_element_type=jnp.float32) / rowsum
    out_ref[:] = jnp.where(hp > 0, hp, jnp.exp(jnp.minimum(hp, 0.0)) - 1.0)


def kernel(input, adj, M, W, a):
    N, din = input.shape
    dout = W.shape[1]
    block_rows = 512
    grid = (N // block_rows,)
    return pl.pallas_call(
        _gat_block_kernel,
        grid=grid,
        in_specs=[
            pl.BlockSpec((N, din), lambda i: (0, 0)),
            pl.BlockSpec((block_rows, din), lambda i: (i, 0)),
            pl.BlockSpec((din, dout), lambda i: (0, 0)),
            pl.BlockSpec((1, 2 * dout), lambda i: (0, 0)),
            pl.BlockSpec((block_rows, N), lambda i: (i, 0)),
            pl.BlockSpec((block_rows, N), lambda i: (i, 0)),
        ],
        out_specs=pl.BlockSpec((block_rows, dout), lambda i: (i, 0)),
        out_shape=jax.ShapeDtypeStruct((N, dout), jnp.float32),
    )(input, input, W, a, adj, M)
